# baseline (speedup 1.0000x reference)
import functools

import jax
import jax.numpy as jnp
from jax import lax
from jax.experimental import pallas as pl
from jax.experimental.pallas import tpu as pltpu
from jax.experimental.pallas import tpu_sc as plsc

NC = 2
NS = 16
NW = NC * NS
L = 16
CH = 128
DEPTH = 4
A_CH = 160


def _make_deg_kernel(kchunks, n_pad):
    rpt = n_pad // NS
    mesh = plsc.VectorSubcoreMesh(core_axis_name="c", subcore_axis_name="s")

    @functools.partial(
        pl.kernel,
        out_type=jax.ShapeDtypeStruct((NC, n_pad), jnp.float32),
        mesh=mesh,
        compiler_params=pltpu.CompilerParams(needs_layout_passes=False),
        scratch_types=[
            pltpu.VMEM((kchunks, CH), jnp.int32),
            pltpu.VMEM((n_pad,), jnp.float32),
            pltpu.VMEM((NS, rpt), jnp.float32),
            pltpu.VMEM((rpt,), jnp.float32),
            pltpu.VMEM_SHARED((NS, n_pad), jnp.float32),
        ],
    )
    def deg_kernel(col_hbm, out_hbm, col_v, hist, rbuf, accv, shared):
        c = lax.axis_index("c")
        s = lax.axis_index("s")
        w = s * NC + c
        off = pl.multiple_of(w * kchunks, 8)
        pltpu.sync_copy(col_hbm.at[pl.ds(off, kchunks)], col_v)
        z16 = jnp.zeros((L,), jnp.float32)

        @pl.loop(0, n_pad // L)
        def _(i):
            hist[pl.ds(i * L, L)] = z16

        ones = jnp.ones((L,), jnp.float32)

        @pl.loop(0, kchunks)
        def _(j):
            for k in range(CH // L):
                idx = col_v[j, pl.ds(k * L, L)]
                plsc.addupdate_scatter(hist, [idx], ones)

        pltpu.sync_copy(hist, shared.at[s])
        plsc.subcore_barrier()
        for r in range(NS):
            pltpu.sync_copy(shared.at[r, pl.ds(s * rpt, rpt)], rbuf.at[r])

        @pl.loop(0, rpt // L)
        def _(v):
            acc = rbuf[0, pl.ds(v * L, L)]
            for r in range(1, NS):
                acc = acc + rbuf[r, pl.ds(v * L, L)]
            accv[pl.ds(v * L, L)] = acc

        pltpu.sync_copy(accv, out_hbm.at[c, pl.ds(s * rpt, rpt)])

    return deg_kernel


def _make_hop_kernel(n_pad, c_dim):
    rpt = n_pad // NS
    mesh = plsc.VectorSubcoreMesh(core_axis_name="c", subcore_axis_name="s")

    @functools.partial(
        pl.kernel,
        out_type=jax.ShapeDtypeStruct((n_pad, c_dim), jnp.float32),
        mesh=mesh,
        compiler_params=pltpu.CompilerParams(needs_layout_passes=False,
                                             use_tc_tiling_on_sc=False),
        scratch_types=[
            pltpu.VMEM((A_CH, CH), jnp.int32),
            pltpu.VMEM((A_CH, CH), jnp.int32),
            pltpu.VMEM_SHARED((n_pad, c_dim), jnp.float32),
        ]
        + [pltpu.VMEM((CH, c_dim), jnp.float32) for _ in range(DEPTH)]
        + [pltpu.SemaphoreType.DMA for _ in range(2 * DEPTH)],
    )
    def hop_kernel(row_hbm, col_hbm, u_hbm, out_hbm,
                   row_v, col_v, accum, *rest):
        gbufs = rest[:DEPTH]
        sems = rest[DEPTH:2 * DEPTH]
        ssems = rest[2 * DEPTH:3 * DEPTH]
        c = lax.axis_index("c")
        s = lax.axis_index("s")

        @pl.when(c == 1)
        def _():
            with jax.named_scope("hop_idx"):
                start = pl.multiple_of(s * A_CH, 8)
                pltpu.sync_copy(row_hbm.at[pl.ds(start, A_CH)], row_v)
                pltpu.sync_copy(col_hbm.at[pl.ds(start, A_CH)], col_v)

            with jax.named_scope("hop_zero"):
                z16 = jnp.zeros((L,), jnp.float32)
                g0 = gbufs[0]

                @pl.loop(0, CH)
                def _(i):
                    for k in range(c_dim // L):
                        g0[i, pl.ds(k * L, L)] = z16

                for k in range(rpt // CH):
                    pltpu.sync_copy(g0, accum.at[pl.ds(s * rpt + k * CH, CH)])
                plsc.subcore_barrier()

            with jax.named_scope("hop_gather"):
                for b in range(DEPTH):
                    pltpu.async_copy(u_hbm.at[row_v.at[b]], gbufs[b], sems[b])

                @pl.loop(0, A_CH, step=DEPTH)
                def _(j):
                    for b in range(DEPTH):
                        pltpu.make_async_copy(u_hbm.at[row_v.at[j + b]],
                                              gbufs[b], sems[b]).wait()
                        pltpu.async_copy(gbufs[b],
                                         accum.at[col_v.at[j + b]],
                                         ssems[b], add=True)
                    for b in range(DEPTH):
                        @pl.when(j + b + DEPTH < A_CH)
                        def _():
                            pltpu.make_async_copy(
                                gbufs[b], accum.at[col_v.at[j + b]],
                                ssems[b]).wait()
                            pltpu.async_copy(
                                u_hbm.at[row_v.at[j + b + DEPTH]],
                                gbufs[b], sems[b])

                for b in range(DEPTH):
                    pltpu.make_async_copy(
                        gbufs[b], accum.at[col_v.at[A_CH - DEPTH + b]],
                        ssems[b]).wait()
                plsc.subcore_barrier()

            with jax.named_scope("hop_wb"):
                pltpu.sync_copy(accum.at[pl.ds(s * rpt, rpt)],
                                out_hbm.at[pl.ds(s * rpt, rpt)])

    return hop_kernel


def _proj_body(x_ref, w_ref, c0_ref, c1_ref, u0_ref, dinv_ref, dsq_ref):
    deg = c0_ref[...] + c1_ref[...] + 2.0
    dinv = lax.rsqrt(deg)
    y = lax.dot_general(x_ref[...], w_ref[...], (((1,), (1,)), ((), ())),
                        preferred_element_type=jnp.float32)
    u0_ref[...] = dinv * y
    dinv_ref[...] = dinv
    dsq_ref[...] = dinv * dinv


def _combine_body(p_ref, u_ref, sc_ref, out_ref):
    out_ref[...] = sc_ref[...] * (p_ref[...] + 2.0 * u_ref[...])


def _final_body(p_ref, u_ref, dinv_ref, b_ref, out_ref):
    logits = dinv_ref[...] * (p_ref[...] + 2.0 * u_ref[...])
    logits = logits + b_ref[...]
    m = jnp.max(logits, axis=1, keepdims=True)
    e = jnp.exp(logits - m)
    lse = jnp.log(jnp.sum(e, axis=1, keepdims=True)) + m
    out_ref[...] = logits - lse


def _row_spec(br, cols):
    return pl.BlockSpec((br, cols), lambda i: (i, 0))


def _full_spec(shape):
    return pl.BlockSpec(shape, lambda i: (0, 0))


def kernel(x, edge_index, W, b):
    n, d = x.shape
    c_dim = W.shape[0]
    e = edge_index.shape[1]

    grain = NS * CH
    n_pad = ((n + grain - 1) // grain) * grain

    proc_chunks = NS * A_CH
    assert proc_chunks * CH >= e, "edge partition must cover all edges"
    pad_chunks = proc_chunks
    kchunks = -(-pad_chunks // (NW * 8)) * 8
    pad_chunks = kchunks * NW
    e_pad = pad_chunks * CH

    pad = jnp.full((e_pad - e,), n, dtype=jnp.int32)
    rowp = jnp.concatenate([edge_index[0], pad]).reshape(pad_chunks, CH)
    colp = jnp.concatenate([edge_index[1], pad]).reshape(pad_chunks, CH)
    x_pad = jnp.pad(x, ((0, n_pad - n), (0, 0)))

    cnt = _make_deg_kernel(kchunks, n_pad)(colp)

    br = 1024
    grid = (n_pad // br,)
    u0, dinv, dsq = pl.pallas_call(
        _proj_body,
        grid=grid,
        in_specs=[_row_spec(br, d), _full_spec((c_dim, d)),
                  _row_spec(br, 1), _row_spec(br, 1)],
        out_specs=[_row_spec(br, c_dim), _row_spec(br, 1), _row_spec(br, 1)],
        out_shape=[jax.ShapeDtypeStruct((n_pad, c_dim), jnp.float32),
                   jax.ShapeDtypeStruct((n_pad, 1), jnp.float32),
                   jax.ShapeDtypeStruct((n_pad, 1), jnp.float32)],
    )(x_pad, W, cnt[0][:, None], cnt[1][:, None])

    hop = _make_hop_kernel(n_pad, c_dim)

    p = hop(rowp, colp, u0)
    u1 = pl.pallas_call(
        _combine_body,
        grid=grid,
        in_specs=[_row_spec(br, c_dim)] * 2 + [_row_spec(br, 1)],
        out_specs=_row_spec(br, c_dim),
        out_shape=jax.ShapeDtypeStruct((n_pad, c_dim), jnp.float32),
    )(p, u0, dsq)

    p2 = hop(rowp, colp, u1)
    out = pl.pallas_call(
        _final_body,
        grid=grid,
        in_specs=[_row_spec(br, c_dim)] * 2 + [_row_spec(br, 1),
                                               _full_spec((1, c_dim))],
        out_specs=_row_spec(br, c_dim),
        out_shape=jax.ShapeDtypeStruct((n_pad, c_dim), jnp.float32),
    )(p2, u1, dinv, b[None, :])

    return out[:n]


# speedup vs baseline: 1.2560x; 1.2560x over previous
import functools

import jax
import jax.numpy as jnp
from jax import lax
from jax.experimental import pallas as pl
from jax.experimental.pallas import tpu as pltpu
from jax.experimental.pallas import tpu_sc as plsc

NC = 2
NS = 16
NW = NC * NS
L = 16
CH = 128
DEPTH = 4
A_CH = 152
B_CH = 8


def _make_deg_kernel(kchunks, n_pad):
    rpt = n_pad // NS
    mesh = plsc.VectorSubcoreMesh(core_axis_name="c", subcore_axis_name="s")

    @functools.partial(
        pl.kernel,
        out_type=jax.ShapeDtypeStruct((NC, n_pad), jnp.float32),
        mesh=mesh,
        compiler_params=pltpu.CompilerParams(needs_layout_passes=False),
        scratch_types=[
            pltpu.VMEM((kchunks, CH), jnp.int32),
            pltpu.VMEM((n_pad,), jnp.float32),
            pltpu.VMEM((NS, rpt), jnp.float32),
            pltpu.VMEM((rpt,), jnp.float32),
            pltpu.VMEM_SHARED((NS, n_pad), jnp.float32),
        ],
    )
    def deg_kernel(col_hbm, out_hbm, col_v, hist, rbuf, accv, shared):
        c = lax.axis_index("c")
        s = lax.axis_index("s")
        w = s * NC + c
        off = pl.multiple_of(w * kchunks, 8)
        pltpu.sync_copy(col_hbm.at[pl.ds(off, kchunks)], col_v)
        z16 = jnp.zeros((L,), jnp.float32)

        @pl.loop(0, n_pad // L)
        def _(i):
            hist[pl.ds(i * L, L)] = z16

        ones = jnp.ones((L,), jnp.float32)

        @pl.loop(0, kchunks)
        def _(j):
            for k in range(CH // L):
                idx = col_v[j, pl.ds(k * L, L)]
                plsc.addupdate_scatter(hist, [idx], ones)

        pltpu.sync_copy(hist, shared.at[s])
        plsc.subcore_barrier()
        for r in range(NS):
            pltpu.sync_copy(shared.at[r, pl.ds(s * rpt, rpt)], rbuf.at[r])

        @pl.loop(0, rpt // L)
        def _(v):
            acc = rbuf[0, pl.ds(v * L, L)]
            for r in range(1, NS):
                acc = acc + rbuf[r, pl.ds(v * L, L)]
            accv[pl.ds(v * L, L)] = acc

        pltpu.sync_copy(accv, out_hbm.at[c, pl.ds(s * rpt, rpt)])

    return deg_kernel


def _make_hop_kernel(n_pad, c_dim):
    rpt = n_pad // NS
    mesh = plsc.VectorSubcoreMesh(core_axis_name="c", subcore_axis_name="s")

    @functools.partial(
        pl.kernel,
        out_type=jax.ShapeDtypeStruct((NC, n_pad, c_dim), jnp.float32),
        mesh=mesh,
        compiler_params=pltpu.CompilerParams(needs_layout_passes=False,
                                             use_tc_tiling_on_sc=False),
        scratch_types=[
            pltpu.VMEM((A_CH, CH), jnp.int32),
            pltpu.VMEM((A_CH, CH), jnp.int32),
            pltpu.VMEM_SHARED((n_pad, c_dim), jnp.float32),
        ]
        + [pltpu.VMEM((CH, c_dim), jnp.float32) for _ in range(DEPTH)]
        + [pltpu.SemaphoreType.DMA for _ in range(DEPTH)],
    )
    def hop_kernel(row_hbm, col_hbm, u_hbm, out_hbm,
                   row_v, col_v, accum, *rest):
        gbufs = rest[:DEPTH]
        sems = rest[DEPTH:2 * DEPTH]
        c = lax.axis_index("c")
        s = lax.axis_index("s")
        start = pl.multiple_of(
            jnp.where(c == 0, s * A_CH, NS * A_CH + s * B_CH), 8)
        my_n = jnp.where(c == 0, A_CH, B_CH)
        pltpu.sync_copy(row_hbm.at[pl.ds(start, A_CH)], row_v)
        pltpu.sync_copy(col_hbm.at[pl.ds(start, A_CH)], col_v)

        z16 = jnp.zeros((L,), jnp.float32)
        g0 = gbufs[0]

        @pl.loop(0, CH)
        def _(i):
            for k in range(c_dim // L):
                g0[i, pl.ds(k * L, L)] = z16

        for k in range(rpt // CH):
            pltpu.sync_copy(g0, accum.at[pl.ds(s * rpt + k * CH, CH)])
        plsc.subcore_barrier()

        for b in range(DEPTH):
            pltpu.async_copy(u_hbm.at[row_v.at[b]], gbufs[b], sems[b])

        @pl.loop(0, my_n, step=DEPTH)
        def _(j):
            for b in range(DEPTH):
                pltpu.make_async_copy(u_hbm.at[row_v.at[j + b]],
                                      gbufs[b], sems[b]).wait()
                pltpu.sync_copy(gbufs[b], accum.at[col_v.at[j + b]],
                                add=True)

                @pl.when(j + b + DEPTH < my_n)
                def _():
                    pltpu.async_copy(u_hbm.at[row_v.at[j + b + DEPTH]],
                                     gbufs[b], sems[b])

        plsc.subcore_barrier()
        pltpu.sync_copy(accum.at[pl.ds(s * rpt, rpt)],
                        out_hbm.at[c, pl.ds(s * rpt, rpt)])

    return hop_kernel


def _proj_body(x_ref, w_ref, c0_ref, c1_ref, u0_ref, dinv_ref, dsq_ref):
    deg = c0_ref[...] + c1_ref[...] + 2.0
    dinv = lax.rsqrt(deg)
    y = lax.dot_general(x_ref[...], w_ref[...], (((1,), (1,)), ((), ())),
                        preferred_element_type=jnp.float32)
    u0_ref[...] = dinv * y
    dinv_ref[...] = dinv
    dsq_ref[...] = dinv * dinv


def _combine_body(p0_ref, p1_ref, u_ref, sc_ref, out_ref):
    out_ref[...] = sc_ref[...] * (p0_ref[...] + p1_ref[...] + 2.0 * u_ref[...])


def _final_body(p0_ref, p1_ref, u_ref, dinv_ref, b_ref, out_ref):
    logits = dinv_ref[...] * (p0_ref[...] + p1_ref[...] + 2.0 * u_ref[...])
    logits = logits + b_ref[...]
    m = jnp.max(logits, axis=1, keepdims=True)
    e = jnp.exp(logits - m)
    lse = jnp.log(jnp.sum(e, axis=1, keepdims=True)) + m
    out_ref[...] = logits - lse


def _row_spec(br, cols):
    return pl.BlockSpec((br, cols), lambda i: (i, 0))


def _full_spec(shape):
    return pl.BlockSpec(shape, lambda i: (0, 0))


def kernel(x, edge_index, W, b):
    n, d = x.shape
    c_dim = W.shape[0]
    e = edge_index.shape[1]

    grain = NS * CH
    n_pad = ((n + grain - 1) // grain) * grain

    proc_chunks = NS * (A_CH + B_CH)
    assert proc_chunks * CH >= e, "edge partition must cover all edges"
    pad_chunks = NS * A_CH + (NS - 1) * B_CH + A_CH
    pad_chunks = max(pad_chunks, proc_chunks)
    kchunks = -(-proc_chunks // (NW * 8)) * 8
    deg_chunks = kchunks * NW
    pad_chunks = max(pad_chunks, deg_chunks)
    pad_chunks = -(-pad_chunks // 8) * 8
    e_pad = pad_chunks * CH

    pad = jnp.full((e_pad - e,), n, dtype=jnp.int32)
    rowp = jnp.concatenate([edge_index[0], pad]).reshape(pad_chunks, CH)
    colp = jnp.concatenate([edge_index[1], pad]).reshape(pad_chunks, CH)
    x_pad = jnp.pad(x, ((0, n_pad - n), (0, 0)))

    cnt = _make_deg_kernel(kchunks, n_pad)(colp[:deg_chunks])

    br = 1024
    grid = (n_pad // br,)
    u0, dinv, dsq = pl.pallas_call(
        _proj_body,
        grid=grid,
        in_specs=[_row_spec(br, d), _full_spec((c_dim, d)),
                  _row_spec(br, 1), _row_spec(br, 1)],
        out_specs=[_row_spec(br, c_dim), _row_spec(br, 1), _row_spec(br, 1)],
        out_shape=[jax.ShapeDtypeStruct((n_pad, c_dim), jnp.float32),
                   jax.ShapeDtypeStruct((n_pad, 1), jnp.float32),
                   jax.ShapeDtypeStruct((n_pad, 1), jnp.float32)],
    )(x_pad, W, cnt[0][:, None], cnt[1][:, None])

    hop = _make_hop_kernel(n_pad, c_dim)

    p = hop(rowp, colp, u0)
    u1 = pl.pallas_call(
        _combine_body,
        grid=grid,
        in_specs=[_row_spec(br, c_dim)] * 3 + [_row_spec(br, 1)],
        out_specs=_row_spec(br, c_dim),
        out_shape=jax.ShapeDtypeStruct((n_pad, c_dim), jnp.float32),
    )(p[0], p[1], u0, dsq)

    p2 = hop(rowp, colp, u1)
    out = pl.pallas_call(
        _final_body,
        grid=grid,
        in_specs=[_row_spec(br, c_dim)] * 3 + [_row_spec(br, 1),
                                               _full_spec((1, c_dim))],
        out_specs=_row_spec(br, c_dim),
        out_shape=jax.ShapeDtypeStruct((n_pad, c_dim), jnp.float32),
    )(p2[0], p2[1], u1, dinv, b[None, :])

    return out[:n]
